# Initial kernel scaffold; baseline (speedup 1.0000x reference)
import functools

import jax
import jax.numpy as jnp
from jax import lax
from jax.experimental import pallas as pl
from jax.experimental.pallas import tpu as pltpu
from jax.experimental.pallas import tpu_sc as plsc

V = 6890
R = 5
A = 8
VP = 6912
NG = VP * R * A
NW = 32
NPW = NG // NW
VPW = NPW // (R * A)


def _make_gather(d, vch):
    ch = vch * R * A
    krows = 3 * ch
    nchunk = VPW // vch
    mesh = plsc.VectorSubcoreMesh(core_axis_name="c", subcore_axis_name="s")

    @functools.partial(
        pl.kernel,
        out_type=jax.ShapeDtypeStruct((VP, R * A * d), jnp.float32),
        mesh=mesh,
        scratch_types=[
            pltpu.VMEM((3 * NPW,), jnp.int32),
            pltpu.VMEM((3 * NPW,), jnp.float32),
            pltpu.VMEM((krows, d), jnp.float32),
            pltpu.VMEM((vch, R * A * d), jnp.float32),
            pltpu.SemaphoreType.DMA,
        ],
    )
    def gather_kernel(table, idx3, w3, out, idx_v, w_v, rows_v, out_v, sem):
        wid = lax.axis_index("s") * 2 + lax.axis_index("c")
        gbase = wid * NPW
        vbase = wid * VPW
        pltpu.sync_copy(idx3.at[pl.ds(3 * gbase, 3 * NPW)], idx_v)
        pltpu.sync_copy(w3.at[pl.ds(3 * gbase, 3 * NPW)], w_v)

        def chunk(t, _):
            pltpu.async_copy(
                table.at[idx_v.at[pl.ds(t * krows, krows)]], rows_v, sem
            ).wait()

            def group(i, _):
                j3 = t * krows + 3 * i
                w0 = plsc.load_gather(w_v, [jnp.full((16,), j3, jnp.int32)])
                w1 = plsc.load_gather(w_v, [jnp.full((16,), j3 + 1, jnp.int32)])
                w2 = plsc.load_gather(w_v, [jnp.full((16,), j3 + 2, jnp.int32)])
                vloc = i // (R * A)
                soff = (i % (R * A)) * d
                for c in range(d // 16):
                    sl = pl.ds(c * 16, 16)
                    acc = (w0 * rows_v[3 * i, sl]
                           + w1 * rows_v[3 * i + 1, sl]
                           + w2 * rows_v[3 * i + 2, sl])
                    out_v[vloc, pl.ds(soff + c * 16, 16)] = acc
                return 0

            lax.fori_loop(0, ch, group, 0)
            pltpu.sync_copy(out_v, out.at[pl.ds(vbase + t * vch, vch)])
            return 0

        lax.fori_loop(0, nchunk, chunk, 0)

    return gather_kernel


def _conv_matmul(interp, w_flat, bias, scale, shift, t_out, vb=256):
    k_dim = interp.shape[1]

    def body(x_ref, w_ref, b_ref, s_ref, be_ref, o_ref):
        x = x_ref[...].astype(jnp.bfloat16)
        z = jnp.dot(x, w_ref[...], preferred_element_type=jnp.float32)
        m = z[:, 0:t_out]
        for rot in range(1, 8):
            m = jnp.maximum(m, z[:, rot * t_out:(rot + 1) * t_out])
        m = m + b_ref[...]
        y = jnp.where(m > 0, m, jnp.expm1(m))
        o_ref[...] = y * s_ref[...] + be_ref[...]

    return pl.pallas_call(
        body,
        grid=(VP // vb,),
        in_specs=[
            pl.BlockSpec((vb, k_dim), lambda i: (i, 0)),
            pl.BlockSpec((k_dim, 8 * t_out), lambda i: (0, 0)),
            pl.BlockSpec((1, t_out), lambda i: (0, 0)),
            pl.BlockSpec((1, t_out), lambda i: (0, 0)),
            pl.BlockSpec((1, t_out), lambda i: (0, 0)),
        ],
        out_specs=pl.BlockSpec((vb, t_out), lambda i: (i, 0)),
        out_shape=jax.ShapeDtypeStruct((VP, t_out), jnp.float32),
    )(interp, w_flat, bias, scale, shift)


def _final_dense(x, w_bf16, bias, rb=256):
    def body(x_ref, w_ref, b_ref, o_ref):
        xb = x_ref[...].astype(jnp.bfloat16)
        o_ref[...] = (
            jnp.dot(xb, w_ref[...], preferred_element_type=jnp.float32)
            + b_ref[...]
        )

    return pl.pallas_call(
        body,
        grid=(VP // rb,),
        in_specs=[
            pl.BlockSpec((rb, x.shape[1]), lambda i: (i, 0)),
            pl.BlockSpec((x.shape[1], V), lambda i: (0, 0)),
            pl.BlockSpec((1, V), lambda i: (0, 0)),
        ],
        out_specs=pl.BlockSpec((rb, V), lambda i: (i, 0)),
        out_shape=jax.ShapeDtypeStruct((V, V), jnp.float32),
    )(x, w_bf16, bias)


def _rot_weights(w, d):
    if w.shape[3] < d:
        w = jnp.pad(w, ((0, 0), (0, 0), (0, 0), (0, d - w.shape[3])))
    t_out = w.shape[0]
    wr = jnp.stack([jnp.roll(w, rot, axis=2) for rot in range(8)], axis=0)
    wr = wr.transpose(2, 3, 4, 0, 1).reshape(R * A * d, 8 * t_out)
    return wr.astype(jnp.bfloat16)


_BN = 1.0 / jnp.sqrt(1.0 + 1e-3)


def kernel(signal, bc, norm_mean, norm_var, W_d0, b_d0, g_d0, be_d0, W_d1, b_d1, g_d1, be_d1, W_m, b_m, g_m, be_m, W_u0, b_u0, g_u0, be_u0, W_u1, b_u1, g_u1, be_u1, W_out, b_out):
    idx = bc[..., 0].astype(jnp.int32).reshape(-1)
    wts = bc[..., 1].reshape(-1)
    pad3 = 3 * NG - idx.shape[0]
    idx3 = jnp.pad(idx, (0, pad3))
    w3 = jnp.pad(wts, (0, pad3))

    s = (signal - norm_mean) / jnp.sqrt(norm_var)
    s16 = jnp.pad(s, ((0, VP - V), (0, 13)))

    g0 = _make_gather(16, 2)
    g128 = _make_gather(128, 2)
    g64 = _make_gather(64, 4)
    g32 = _make_gather(32, 4)

    def conv(table, gfn, w, b, g, be, d):
        interp = gfn(table, idx3, w3)
        return _conv_matmul(
            interp, _rot_weights(w, d), b[None, :],
            (g * _BN)[None, :], be[None, :], w.shape[0])

    s0 = conv(s16, g0, W_d0, b_d0, g_d0, be_d0, 16)
    s1 = conv(s0, g128, W_d1, b_d1, g_d1, be_d1, 128)
    m = conv(s1, g64, W_m, b_m, g_m, be_m, 64)
    u0 = conv(m, g32, W_u0, b_u0, g_u0, be_u0, 32)
    u0c = jnp.concatenate([u0, s1], axis=-1)
    u1 = conv(u0c, g128, W_u1, b_u1, g_u1, be_u1, 128)
    u1c = jnp.concatenate([u1, s0], axis=-1)

    return _final_dense(u1c, W_out.astype(jnp.bfloat16), b_out[None, :])


# speedup vs baseline: 4.1949x; 4.1949x over previous
import functools

import jax
import jax.numpy as jnp
from jax import lax
from jax.experimental import pallas as pl
from jax.experimental.pallas import tpu as pltpu
from jax.experimental.pallas import tpu_sc as plsc

V = 6890
R = 5
A = 8
VP = 6912
NG = VP * R * A
NW = 32
NPW = NG // NW
VPW = NPW // (R * A)


def _make_gather(d, vch):
    ch = vch * R * A
    krows = 3 * ch
    nchunk = VPW // vch
    mesh = plsc.VectorSubcoreMesh(core_axis_name="c", subcore_axis_name="s")

    @functools.partial(
        pl.kernel,
        out_type=jax.ShapeDtypeStruct((VP, R * A * d), jnp.float32),
        mesh=mesh,
        scratch_types=[
            pltpu.VMEM((krows,), jnp.int32),
            pltpu.VMEM((krows + 16,), jnp.float32),
            pltpu.VMEM((krows, d), jnp.float32),
            pltpu.VMEM((vch, R * A * d), jnp.float32),
            pltpu.SemaphoreType.DMA,
        ],
        compiler_params=pltpu.CompilerParams(use_tc_tiling_on_sc=False),
    )
    def gather_kernel(table, idx3, w3, out, idx_c, w_c, rows_v, out_v, sem):
        wid = lax.axis_index("s") * 2 + lax.axis_index("c")
        gbase = wid * NPW
        vbase = wid * VPW

        def chunk(t, _):
            pltpu.sync_copy(idx3.at[pl.ds(3 * gbase + t * krows, krows)], idx_c)
            pltpu.sync_copy(w3.at[pl.ds(3 * gbase + t * krows, krows)],
                            w_c.at[pl.ds(0, krows)])
            pltpu.async_copy(table.at[idx_c], rows_v, sem).wait()

            def group(i, _):
                wv = w_c[pl.ds(3 * i, 16)]
                w0 = wv[0]
                w1 = wv[1]
                w2 = wv[2]
                vloc = i // (R * A)
                soff = (i % (R * A)) * d
                for c in range(d // 16):
                    sl = pl.ds(c * 16, 16)
                    acc = (w0 * rows_v[3 * i, sl]
                           + w1 * rows_v[3 * i + 1, sl]
                           + w2 * rows_v[3 * i + 2, sl])
                    out_v[vloc, pl.ds(soff + c * 16, 16)] = acc
                return 0

            lax.fori_loop(0, ch, group, 0)
            pltpu.sync_copy(out_v, out.at[pl.ds(vbase + t * vch, vch)])
            return 0

        lax.fori_loop(0, nchunk, chunk, 0)

    return gather_kernel


def _conv_matmul(interp, w_flat, bias, scale, shift, t_out, vb=256):
    k_dim = interp.shape[1]

    def body(x_ref, w_ref, b_ref, s_ref, be_ref, o_ref):
        x = x_ref[...].astype(jnp.bfloat16)
        z = jnp.dot(x, w_ref[...], preferred_element_type=jnp.float32)
        m = z[:, 0:t_out]
        for rot in range(1, 8):
            m = jnp.maximum(m, z[:, rot * t_out:(rot + 1) * t_out])
        m = m + b_ref[...]
        y = jnp.where(m > 0, m, jnp.exp(m) - 1.0)
        o_ref[...] = y * s_ref[...] + be_ref[...]

    return pl.pallas_call(
        body,
        grid=(VP // vb,),
        in_specs=[
            pl.BlockSpec((vb, k_dim), lambda i: (i, 0)),
            pl.BlockSpec((k_dim, 8 * t_out), lambda i: (0, 0)),
            pl.BlockSpec((1, t_out), lambda i: (0, 0)),
            pl.BlockSpec((1, t_out), lambda i: (0, 0)),
            pl.BlockSpec((1, t_out), lambda i: (0, 0)),
        ],
        out_specs=pl.BlockSpec((vb, t_out), lambda i: (i, 0)),
        out_shape=jax.ShapeDtypeStruct((VP, t_out), jnp.float32),
    )(interp, w_flat, bias, scale, shift)


def _final_dense(x, w_bf16, bias, rb=256):
    def body(x_ref, w_ref, b_ref, o_ref):
        xb = x_ref[...].astype(jnp.bfloat16)
        o_ref[...] = (
            jnp.dot(xb, w_ref[...], preferred_element_type=jnp.float32)
            + b_ref[...]
        )

    return pl.pallas_call(
        body,
        grid=(VP // rb,),
        in_specs=[
            pl.BlockSpec((rb, x.shape[1]), lambda i: (i, 0)),
            pl.BlockSpec((x.shape[1], V), lambda i: (0, 0)),
            pl.BlockSpec((1, V), lambda i: (0, 0)),
        ],
        out_specs=pl.BlockSpec((rb, V), lambda i: (i, 0)),
        out_shape=jax.ShapeDtypeStruct((V, V), jnp.float32),
    )(x, w_bf16, bias)


def _rot_weights(w, d):
    if w.shape[3] < d:
        w = jnp.pad(w, ((0, 0), (0, 0), (0, 0), (0, d - w.shape[3])))
    t_out = w.shape[0]
    wr = jnp.stack([jnp.roll(w, rot, axis=2) for rot in range(8)], axis=0)
    wr = wr.transpose(2, 3, 4, 0, 1).reshape(R * A * d, 8 * t_out)
    return wr.astype(jnp.bfloat16)


_BN = (1.0 + 1e-3) ** -0.5


def kernel(signal, bc, norm_mean, norm_var, W_d0, b_d0, g_d0, be_d0, W_d1, b_d1, g_d1, be_d1, W_m, b_m, g_m, be_m, W_u0, b_u0, g_u0, be_u0, W_u1, b_u1, g_u1, be_u1, W_out, b_out):
    idx = bc[..., 0].astype(jnp.int32).reshape(-1)
    wts = bc[..., 1].reshape(-1)
    pad3 = 3 * NG - idx.shape[0]
    idx3 = jnp.pad(idx, (0, pad3))
    w3 = jnp.pad(wts, (0, pad3))

    s = (signal - norm_mean) / jnp.sqrt(norm_var)
    s16 = jnp.pad(s, ((0, VP - V), (0, 13)))

    g0 = _make_gather(16, 2)
    g128 = _make_gather(128, 2)
    g64 = _make_gather(64, 4)
    g32 = _make_gather(32, 4)

    def conv(table, gfn, w, b, g, be, d):
        interp = gfn(table, idx3, w3)
        return _conv_matmul(
            interp, _rot_weights(w, d), b[None, :],
            (g * _BN)[None, :], be[None, :], w.shape[0])

    s0 = conv(s16, g0, W_d0, b_d0, g_d0, be_d0, 16)
    s1 = conv(s0, g128, W_d1, b_d1, g_d1, be_d1, 128)
    m = conv(s1, g64, W_m, b_m, g_m, be_m, 64)
    u0 = conv(m, g32, W_u0, b_u0, g_u0, be_u0, 32)
    u0c = jnp.concatenate([u0, s1], axis=-1)
    u1 = conv(u0c, g128, W_u1, b_u1, g_u1, be_u1, 128)
    u1c = jnp.concatenate([u1, s0], axis=-1)

    return _final_dense(u1c, W_out.astype(jnp.bfloat16), b_out[None, :])


# speedup vs baseline: 5.7155x; 1.3625x over previous
import functools

import jax
import jax.numpy as jnp
from jax import lax
from jax.experimental import pallas as pl
from jax.experimental.pallas import tpu as pltpu
from jax.experimental.pallas import tpu_sc as plsc

V = 6890
R = 5
A = 8
VP = 6912
NG = VP * R * A
NW = 32
NPW = NG // NW
VPW = NPW // (R * A)


def _make_gather(d, vch):
    ch = vch * R * A
    krows = 3 * ch
    nchunk = VPW // vch
    mesh = plsc.VectorSubcoreMesh(core_axis_name="c", subcore_axis_name="s")

    @functools.partial(
        pl.kernel,
        out_type=jax.ShapeDtypeStruct((VP, R * A * d), jnp.float32),
        mesh=mesh,
        scratch_types=[
            pltpu.VMEM((3 * NPW + krows,), jnp.int32),
            pltpu.VMEM((krows + 16,), jnp.float32),
            pltpu.VMEM((krows + 16,), jnp.float32),
            pltpu.VMEM((krows, d), jnp.float32),
            pltpu.VMEM((krows, d), jnp.float32),
            pltpu.VMEM((vch, R * A * d), jnp.float32),
            pltpu.SemaphoreType.DMA,
            pltpu.SemaphoreType.DMA,
            pltpu.SemaphoreType.DMA,
            pltpu.SemaphoreType.DMA,
        ],
        compiler_params=pltpu.CompilerParams(use_tc_tiling_on_sc=False),
    )
    def gather_kernel(table, idx3, w3, out, idx_v, w_c0, w_c1, rows0, rows1,
                      out_v, semg0, semg1, semw0, semw1):
        wid = lax.axis_index("s") * 2 + lax.axis_index("c")
        gbase = wid * NPW
        vbase = wid * VPW
        pltpu.sync_copy(idx3.at[pl.ds(3 * gbase, 3 * NPW + krows)], idx_v)

        w_c = (w_c0, w_c1)
        rows = (rows0, rows1)
        semg = (semg0, semg1)
        semw = (semw0, semw1)

        def start(t, b):
            pltpu.async_copy(w3.at[pl.ds(3 * gbase + t * krows, krows)],
                             w_c[b].at[pl.ds(0, krows)], semw[b])
            pltpu.async_copy(table.at[idx_v.at[pl.ds(t * krows, krows)]],
                             rows[b], semg[b])

        def compute(t, b):
            rows_v = rows[b]
            w_v = w_c[b]

            def group(i, _):
                wv = w_v[pl.ds(3 * i, 16)]
                w0 = wv[0]
                w1 = wv[1]
                w2 = wv[2]
                vloc = i // (R * A)
                soff = (i % (R * A)) * d
                for c in range(d // 16):
                    sl = pl.ds(c * 16, 16)
                    acc = (w0 * rows_v[3 * i, sl]
                           + w1 * rows_v[3 * i + 1, sl]
                           + w2 * rows_v[3 * i + 2, sl])
                    out_v[vloc, pl.ds(soff + c * 16, 16)] = acc
                return 0

            lax.fori_loop(0, ch, group, 0, unroll=2)
            pltpu.sync_copy(out_v, out.at[pl.ds(vbase + t * vch, vch)])

        def wait(b):
            pltpu.make_async_copy(w3.at[pl.ds(0, krows)],
                                  w_c[b].at[pl.ds(0, krows)], semw[b]).wait()
            pltpu.make_async_copy(table.at[idx_v.at[pl.ds(0, krows)]],
                                  rows[b], semg[b]).wait()

        start(0, 0)

        def step(p, _):
            t = 2 * p
            start(t + 1, 1)
            wait(0)
            compute(t, 0)
            start(t + 2, 0)
            wait(1)
            compute(t + 1, 1)
            return 0

        lax.fori_loop(0, nchunk // 2, step, 0)
        wait(0)

    return gather_kernel


def _conv_matmul(interp, w_flat, bias, scale, shift, t_out, vb=256):
    k_dim = interp.shape[1]

    def body(x_ref, w_ref, b_ref, s_ref, be_ref, o_ref):
        x = x_ref[...].astype(jnp.bfloat16)
        z = jnp.dot(x, w_ref[...], preferred_element_type=jnp.float32)
        m = z[:, 0:t_out]
        for rot in range(1, 8):
            m = jnp.maximum(m, z[:, rot * t_out:(rot + 1) * t_out])
        m = m + b_ref[...]
        y = jnp.where(m > 0, m, jnp.exp(m) - 1.0)
        o_ref[...] = y * s_ref[...] + be_ref[...]

    return pl.pallas_call(
        body,
        grid=(VP // vb,),
        in_specs=[
            pl.BlockSpec((vb, k_dim), lambda i: (i, 0)),
            pl.BlockSpec((k_dim, 8 * t_out), lambda i: (0, 0)),
            pl.BlockSpec((1, t_out), lambda i: (0, 0)),
            pl.BlockSpec((1, t_out), lambda i: (0, 0)),
            pl.BlockSpec((1, t_out), lambda i: (0, 0)),
        ],
        out_specs=pl.BlockSpec((vb, t_out), lambda i: (i, 0)),
        out_shape=jax.ShapeDtypeStruct((VP, t_out), jnp.float32),
    )(interp, w_flat, bias, scale, shift)


def _final_dense(x, w_bf16, bias, rb=256):
    def body(x_ref, w_ref, b_ref, o_ref):
        xb = x_ref[...].astype(jnp.bfloat16)
        o_ref[...] = (
            jnp.dot(xb, w_ref[...], preferred_element_type=jnp.float32)
            + b_ref[...]
        )

    return pl.pallas_call(
        body,
        grid=(VP // rb,),
        in_specs=[
            pl.BlockSpec((rb, x.shape[1]), lambda i: (i, 0)),
            pl.BlockSpec((x.shape[1], V), lambda i: (0, 0)),
            pl.BlockSpec((1, V), lambda i: (0, 0)),
        ],
        out_specs=pl.BlockSpec((rb, V), lambda i: (i, 0)),
        out_shape=jax.ShapeDtypeStruct((V, V), jnp.float32),
    )(x, w_bf16, bias)


def _rot_weights(w, d):
    if w.shape[3] < d:
        w = jnp.pad(w, ((0, 0), (0, 0), (0, 0), (0, d - w.shape[3])))
    t_out = w.shape[0]
    wr = jnp.stack([jnp.roll(w, rot, axis=2) for rot in range(8)], axis=0)
    wr = wr.transpose(2, 3, 4, 0, 1).reshape(R * A * d, 8 * t_out)
    return wr.astype(jnp.bfloat16)


_BN = (1.0 + 1e-3) ** -0.5


def kernel(signal, bc, norm_mean, norm_var, W_d0, b_d0, g_d0, be_d0, W_d1, b_d1, g_d1, be_d1, W_m, b_m, g_m, be_m, W_u0, b_u0, g_u0, be_u0, W_u1, b_u1, g_u1, be_u1, W_out, b_out):
    idx = bc[..., 0].astype(jnp.int32).reshape(-1)
    wts = bc[..., 1].reshape(-1)
    pad3 = 3 * NG + 1440 - idx.shape[0]
    idx3 = jnp.pad(idx, (0, pad3))
    w3 = jnp.pad(wts, (0, pad3))

    s = (signal - norm_mean) / jnp.sqrt(norm_var)
    s16 = jnp.pad(s, ((0, VP - V), (0, 13)))

    g0 = _make_gather(16, 4)
    g128 = _make_gather(128, 2)
    g64 = _make_gather(64, 4)
    g32 = _make_gather(32, 4)

    def conv(table, gfn, w, b, g, be, d):
        interp = gfn(table, idx3, w3)
        return _conv_matmul(
            interp, _rot_weights(w, d), b[None, :],
            (g * _BN)[None, :], be[None, :], w.shape[0])

    s0 = conv(s16, g0, W_d0, b_d0, g_d0, be_d0, 16)
    s1 = conv(s0, g128, W_d1, b_d1, g_d1, be_d1, 128)
    m = conv(s1, g64, W_m, b_m, g_m, be_m, 64)
    u0 = conv(m, g32, W_u0, b_u0, g_u0, be_u0, 32)
    u0c = jnp.concatenate([u0, s1], axis=-1)
    u1 = conv(u0c, g128, W_u1, b_u1, g_u1, be_u1, 128)
    u1c = jnp.concatenate([u1, s0], axis=-1)

    return _final_dense(u1c, W_out.astype(jnp.bfloat16), b_out[None, :])
